# Initial kernel scaffold; baseline (speedup 1.0000x reference)
import functools

import jax
import jax.numpy as jnp
from jax import lax
from jax.experimental import pallas as pl
from jax.experimental.pallas import tpu as pltpu
from jax.experimental.pallas import tpu_sc as plsc

N = 10000
D = 128
DE = 4
NC = 2
NS = 16
NW = NC * NS
B = 128
NCHUNK = D // 16

ROWS_PER_TILE = 640
N_PAD = NS * ROWS_PER_TILE


def _sc_edge_kernel(e_pad: int):
    batches_per_worker = e_pad // (NW * B)
    mesh = plsc.VectorSubcoreMesh(
        core_axis_name="c", subcore_axis_name="s", num_cores=NC, num_subcores=NS
    )

    @functools.partial(
        pl.kernel,
        out_type=jax.ShapeDtypeStruct((NC * N_PAD, D), jnp.float32),
        mesh=mesh,
        scratch_types=[
            pltpu.VMEM((B,), jnp.int32),
            pltpu.VMEM((B,), jnp.int32),
            pltpu.VMEM((DE, B), jnp.float32),
            pltpu.VMEM((B, D), jnp.float32),
            pltpu.VMEM((DE, D), jnp.float32),
            pltpu.VMEM((D,), jnp.float32),
            pltpu.VMEM((B, D), jnp.float32),
            pltpu.VMEM_SHARED((N_PAD, D), jnp.float32),
            pltpu.SemaphoreType.DMA,
        ],
    )
    def k(x_hbm, src_hbm, dst_hbm, attr_hbm, we_hbm, be_hbm, out_hbm,
          src_v, dst_v, attr_v, gbuf, we_v, be_v, obuf, accum, gsem):
        cid = lax.axis_index("c")
        tid = lax.axis_index("s")
        wid = tid * NC + cid

        pltpu.sync_copy(we_hbm, we_v)
        pltpu.sync_copy(be_hbm, be_v)

        zero16 = jnp.zeros((16,), jnp.float32)

        def zrow(i, _):
            for c in range(NCHUNK):
                obuf[i, pl.ds(c * 16, 16)] = zero16
            return 0

        lax.fori_loop(0, B, zrow, 0)
        for j in range(ROWS_PER_TILE // B):
            pltpu.sync_copy(obuf, accum.at[pl.ds(tid * ROWS_PER_TILE + j * B, B)])
        plsc.subcore_barrier()

        wv = [[we_v[kk, pl.ds(c * 16, 16)] for kk in range(DE)]
              for c in range(NCHUNK)]
        bv = [be_v[pl.ds(c * 16, 16)] for c in range(NCHUNK)]

        def edge_body(e, _):
            a0 = attr_v[0, e]
            a1 = attr_v[1, e]
            a2 = attr_v[2, e]
            a3 = attr_v[3, e]
            for c in range(NCHUNK):
                g = gbuf[e, pl.ds(c * 16, 16)]
                m = (g + bv[c] + a0 * wv[c][0] + a1 * wv[c][1]
                     + a2 * wv[c][2] + a3 * wv[c][3])
                gbuf[e, pl.ds(c * 16, 16)] = jnp.maximum(m, 0.0)
            return 0

        def batch_body(b, _):
            bid = wid * batches_per_worker + b
            base = bid * B
            pltpu.sync_copy(src_hbm.at[pl.ds(base, B)], src_v)
            pltpu.sync_copy(dst_hbm.at[pl.ds(base, B)], dst_v)
            pltpu.sync_copy(attr_hbm.at[bid], attr_v)
            pltpu.async_copy(x_hbm.at[src_v], gbuf, gsem).wait()
            lax.fori_loop(0, B, edge_body, 0)
            pltpu.sync_copy(gbuf, accum.at[dst_v], add=True)
            return 0

        lax.fori_loop(0, batches_per_worker, batch_body, 0)
        plsc.subcore_barrier()

        for j in range(ROWS_PER_TILE // B):
            off = tid * ROWS_PER_TILE + j * B
            pltpu.sync_copy(accum.at[pl.ds(off, B)], obuf)
            pltpu.sync_copy(obuf, out_hbm.at[pl.ds(cid * N_PAD + off, B)])

    return k


def _mlp_kernel(p0_ref, p1_ref, x_ref, w1_ref, b1_ref, w2_ref, b2_ref, o_ref):
    h = p0_ref[...] + p1_ref[...] + x_ref[...]
    h1 = jnp.maximum(
        jnp.dot(h, w1_ref[...], preferred_element_type=jnp.float32)
        + b1_ref[...], 0.0)
    o_ref[...] = (jnp.dot(h1, w2_ref[...], preferred_element_type=jnp.float32)
                  + b2_ref[...])


def kernel(x, edge_index, edge_attr, We, be, W1, b1, W2, b2):
    e = edge_index.shape[1]
    e_pad = ((e + NW * B - 1) // (NW * B)) * (NW * B)
    src = edge_index[0].astype(jnp.int32)
    dst = edge_index[1].astype(jnp.int32)
    pad = e_pad - e
    src = jnp.pad(src, (0, pad))
    dst = jnp.pad(dst, (0, pad), constant_values=N)
    attr = jnp.pad(edge_attr.astype(jnp.float32), ((0, pad), (0, 0)))
    attr3 = attr.reshape(e_pad // B, B, DE).transpose(0, 2, 1)

    parts = _sc_edge_kernel(e_pad)(x, src, dst, attr3, We, be)

    p0 = parts[:N]
    p1 = parts[N_PAD:N_PAD + N]

    blk = 1000
    out = pl.pallas_call(
        _mlp_kernel,
        grid=(N // blk,),
        in_specs=[
            pl.BlockSpec((blk, D), lambda i: (i, 0)),
            pl.BlockSpec((blk, D), lambda i: (i, 0)),
            pl.BlockSpec((blk, D), lambda i: (i, 0)),
            pl.BlockSpec((D, D), lambda i: (0, 0)),
            pl.BlockSpec((1, D), lambda i: (0, 0)),
            pl.BlockSpec((D, D), lambda i: (0, 0)),
            pl.BlockSpec((1, D), lambda i: (0, 0)),
        ],
        out_specs=pl.BlockSpec((blk, D), lambda i: (i, 0)),
        out_shape=jax.ShapeDtypeStruct((N, D), jnp.float32),
    )(p0, p1, x, W1, b1.reshape(1, D), W2, b2.reshape(1, D))
    return out


# speedup vs baseline: 2.5527x; 2.5527x over previous
import functools

import jax
import jax.numpy as jnp
from jax import lax
from jax.experimental import pallas as pl
from jax.experimental.pallas import tpu as pltpu
from jax.experimental.pallas import tpu_sc as plsc

N = 10000
D = 128
DE = 4
NC = 2
NS = 16
NW = NC * NS
B = 128
NCHUNK = D // 16

ROWS_PER_TILE = 640
N_PAD = NS * ROWS_PER_TILE


def _sc_edge_kernel(e_pad: int):
    batches_per_worker = e_pad // (NW * B)
    mesh = plsc.VectorSubcoreMesh(
        core_axis_name="c", subcore_axis_name="s", num_cores=NC, num_subcores=NS
    )

    @functools.partial(
        pl.kernel,
        out_type=jax.ShapeDtypeStruct((NC * N_PAD, D), jnp.float32),
        mesh=mesh,
        scratch_types=[
            pltpu.VMEM((B,), jnp.int32),
            pltpu.VMEM((B,), jnp.int32),
            pltpu.VMEM((DE, B), jnp.float32),
            pltpu.VMEM((B, D), jnp.float32),
            pltpu.VMEM((DE, D), jnp.float32),
            pltpu.VMEM((D,), jnp.float32),
            pltpu.VMEM((B, D), jnp.float32),
            pltpu.VMEM_SHARED((N_PAD, D), jnp.float32),
            pltpu.SemaphoreType.DMA,
        ],
    )
    def k(x_hbm, src_hbm, dst_hbm, attr_hbm, we_hbm, be_hbm, out_hbm,
          src_v, dst_v, attr_v, gbuf, we_v, be_v, obuf, accum, gsem):
        cid = lax.axis_index("c")
        tid = lax.axis_index("s")
        wid = tid * NC + cid

        pltpu.sync_copy(we_hbm, we_v)
        pltpu.sync_copy(be_hbm, be_v)

        zero16 = jnp.zeros((16,), jnp.float32)

        def zrow(i, _):
            for c in range(NCHUNK):
                obuf[i, pl.ds(c * 16, 16)] = zero16
            return 0

        lax.fori_loop(0, B, zrow, 0)
        for j in range(ROWS_PER_TILE // B):
            pltpu.sync_copy(obuf, accum.at[pl.ds(tid * ROWS_PER_TILE + j * B, B)])
        plsc.subcore_barrier()

        wv = [[we_v[kk, pl.ds(c * 16, 16)] for kk in range(DE)]
              for c in range(NCHUNK)]
        bv = [be_v[pl.ds(c * 16, 16)] for c in range(NCHUNK)]

        def group_body(g, _):
            av = [attr_v[kk, pl.ds(g * 16, 16)] for kk in range(DE)]
            for j in range(16):
                e = g * 16 + j
                a = [av[kk][j] for kk in range(DE)]
                for c in range(NCHUNK):
                    gv = gbuf[e, pl.ds(c * 16, 16)]
                    m = (gv + bv[c] + a[0] * wv[c][0] + a[1] * wv[c][1]
                         + a[2] * wv[c][2] + a[3] * wv[c][3])
                    gbuf[e, pl.ds(c * 16, 16)] = jnp.maximum(m, 0.0)
            return 0

        def batch_body(b, _):
            bid = wid * batches_per_worker + b
            base = bid * B
            pltpu.sync_copy(src_hbm.at[pl.ds(base, B)], src_v)
            pltpu.sync_copy(dst_hbm.at[pl.ds(base, B)], dst_v)
            pltpu.sync_copy(attr_hbm.at[bid], attr_v)
            pltpu.async_copy(x_hbm.at[src_v], gbuf, gsem).wait()
            lax.fori_loop(0, B // 16, group_body, 0)
            pltpu.sync_copy(gbuf, accum.at[dst_v], add=True)
            return 0

        lax.fori_loop(0, batches_per_worker, batch_body, 0)
        plsc.subcore_barrier()

        for j in range(ROWS_PER_TILE // B):
            off = tid * ROWS_PER_TILE + j * B
            pltpu.sync_copy(accum.at[pl.ds(off, B)], obuf)
            pltpu.sync_copy(obuf, out_hbm.at[pl.ds(cid * N_PAD + off, B)])

    return k


def _mlp_kernel(p0_ref, p1_ref, x_ref, w1_ref, b1_ref, w2_ref, b2_ref, o_ref):
    h = p0_ref[...] + p1_ref[...] + x_ref[...]
    h1 = jnp.maximum(
        jnp.dot(h, w1_ref[...], preferred_element_type=jnp.float32)
        + b1_ref[...], 0.0)
    o_ref[...] = (jnp.dot(h1, w2_ref[...], preferred_element_type=jnp.float32)
                  + b2_ref[...])


def kernel(x, edge_index, edge_attr, We, be, W1, b1, W2, b2):
    e = edge_index.shape[1]
    e_pad = ((e + NW * B - 1) // (NW * B)) * (NW * B)
    src = edge_index[0].astype(jnp.int32)
    dst = edge_index[1].astype(jnp.int32)
    pad = e_pad - e
    src = jnp.pad(src, (0, pad))
    dst = jnp.pad(dst, (0, pad), constant_values=N)
    attr = jnp.pad(edge_attr.astype(jnp.float32), ((0, pad), (0, 0)))
    attr3 = attr.reshape(e_pad // B, B, DE).transpose(0, 2, 1)

    parts = _sc_edge_kernel(e_pad)(x, src, dst, attr3, We, be)

    p0 = parts[:N]
    p1 = parts[N_PAD:N_PAD + N]

    blk = 1000
    out = pl.pallas_call(
        _mlp_kernel,
        grid=(N // blk,),
        in_specs=[
            pl.BlockSpec((blk, D), lambda i: (i, 0)),
            pl.BlockSpec((blk, D), lambda i: (i, 0)),
            pl.BlockSpec((blk, D), lambda i: (i, 0)),
            pl.BlockSpec((D, D), lambda i: (0, 0)),
            pl.BlockSpec((1, D), lambda i: (0, 0)),
            pl.BlockSpec((D, D), lambda i: (0, 0)),
            pl.BlockSpec((1, D), lambda i: (0, 0)),
        ],
        out_specs=pl.BlockSpec((blk, D), lambda i: (i, 0)),
        out_shape=jax.ShapeDtypeStruct((N, D), jnp.float32),
    )(p0, p1, x, W1, b1.reshape(1, D), W2, b2.reshape(1, D))
    return out
